# baseline (speedup 1.0000x reference)
import jax
import jax.numpy as jnp
from jax import lax
from jax.experimental import pallas as pl
from jax.experimental.pallas import tpu as pltpu
from jax.experimental.pallas import tpu_sc as plsc

B = 16384
ITEM_VOCAB = 1000000
D_ITEM = 32
D_GROUP = 16
H1, H2 = 128, 64

NC, NS = 2, 16
NW = NC * NS
BPW = B // NW

BLK = 2048


def _gather_body(fidx_hbm, gh_hbm, gn_hbm, item_flat, gh_t, gn_t,
                 item_out, gh_out, gn_out,
                 fidx_v, gh_v, gn_v, item_vals, gh_rows, gn_rows,
                 sem1, sem2, sem3):
    wid = lax.axis_index("s") * NC + lax.axis_index("c")
    base = wid * BPW
    pltpu.sync_copy(fidx_hbm.at[pl.ds(base * D_ITEM, BPW * D_ITEM)], fidx_v)
    pltpu.sync_copy(gh_hbm.at[pl.ds(base, BPW)], gh_v)
    pltpu.sync_copy(gn_hbm.at[pl.ds(base, BPW)], gn_v)
    c1 = pltpu.async_copy(item_flat.at[fidx_v], item_vals, sem1)
    c2 = pltpu.async_copy(gh_t.at[gh_v], gh_rows, sem2)
    c3 = pltpu.async_copy(gn_t.at[gn_v], gn_rows, sem3)
    c1.wait()
    c2.wait()
    c3.wait()
    pltpu.sync_copy(item_vals, item_out.at[pl.ds(base * D_ITEM, BPW * D_ITEM)])
    pltpu.sync_copy(gh_rows, gh_out.at[pl.ds(base, BPW)])
    pltpu.sync_copy(gn_rows, gn_out.at[pl.ds(base, BPW)])


def _sc_gather():
    return pl.kernel(
        _gather_body,
        out_type=[
            jax.ShapeDtypeStruct((B * D_ITEM,), jnp.float32),
            jax.ShapeDtypeStruct((B, D_GROUP), jnp.float32),
            jax.ShapeDtypeStruct((B, D_GROUP), jnp.float32),
        ],
        mesh=plsc.VectorSubcoreMesh(
            core_axis_name="c", subcore_axis_name="s",
            num_cores=NC, num_subcores=NS),
        compiler_params=pltpu.CompilerParams(use_tc_tiling_on_sc=False),
        scratch_types=[
            pltpu.VMEM((BPW * D_ITEM,), jnp.int32),
            pltpu.VMEM((BPW,), jnp.int32),
            pltpu.VMEM((BPW,), jnp.int32),
            pltpu.VMEM((BPW * D_ITEM,), jnp.float32),
            pltpu.VMEM((BPW, D_GROUP), jnp.float32),
            pltpu.VMEM((BPW, D_GROUP), jnp.float32),
            pltpu.SemaphoreType.DMA,
            pltpu.SemaphoreType.DMA,
            pltpu.SemaphoreType.DMA,
        ],
    )


def _mlp_body(item_ref, gh_ref, gn_ref, pb_ref,
              w1a_ref, w1b_ref, w1c_ref, w1pq_ref, b1_ref,
              w2_ref, b2_ref, wp_ref, bp_ref, out_ref):
    h = jnp.dot(item_ref[...], w1a_ref[...], preferred_element_type=jnp.float32)
    h += jnp.dot(gh_ref[...], w1b_ref[...], preferred_element_type=jnp.float32)
    h += jnp.dot(gn_ref[...], w1c_ref[...], preferred_element_type=jnp.float32)
    h += jnp.dot(pb_ref[...], w1pq_ref[...], preferred_element_type=jnp.float32)
    h = jnp.maximum(h + b1_ref[...], 0.0)
    h = jnp.maximum(
        jnp.dot(h, w2_ref[...], preferred_element_type=jnp.float32) + b2_ref[...],
        0.0)
    out_ref[...] = (
        jnp.dot(h, wp_ref[...], preferred_element_type=jnp.float32) + bp_ref[...])


def _mlp(item_vec, gh_vec, gn_vec, pb, W1a, W1b, W1c, W1pq, b1, W2, b2,
         Wp, bp, interpret=False):
    row = lambda i: (i, 0)
    rep = lambda i: (0, 0)
    return pl.pallas_call(
        _mlp_body,
        grid=(B // BLK,),
        in_specs=[
            pl.BlockSpec((BLK, D_ITEM), row),
            pl.BlockSpec((BLK, D_GROUP), row),
            pl.BlockSpec((BLK, D_GROUP), row),
            pl.BlockSpec((BLK, 2), row),
            pl.BlockSpec((D_ITEM, H1), rep),
            pl.BlockSpec((D_GROUP, H1), rep),
            pl.BlockSpec((D_GROUP, H1), rep),
            pl.BlockSpec((2, H1), rep),
            pl.BlockSpec((1, H1), rep),
            pl.BlockSpec((H1, H2), rep),
            pl.BlockSpec((1, H2), rep),
            pl.BlockSpec((H2, H2), rep),
            pl.BlockSpec((1, H2), rep),
        ],
        out_specs=pl.BlockSpec((BLK, H2), row),
        out_shape=jax.ShapeDtypeStruct((B, H2), jnp.float32),
        interpret=interpret,
    )(item_vec, gh_vec, gn_vec, pb, W1a, W1b, W1c, W1pq, b1, W2, b2, Wp, bp)


def kernel(ProductCode, ProductGroupHeader, ProductGroupName, Price, IsBestSeller,
           item_table, gh_table, gn_table, W1, b1, W2, b2, Wp, bp):
    tbl_flat = item_table.T.reshape(ITEM_VOCAB * D_ITEM)
    flat_idx = (ProductCode[:, None]
                + jnp.arange(D_ITEM, dtype=jnp.int32)[None, :] * ITEM_VOCAB
                ).reshape(B * D_ITEM)
    item_flat, gh_vec, gn_vec = _sc_gather()(
        flat_idx, ProductGroupHeader, ProductGroupName,
        tbl_flat, gh_table, gn_table)
    item_vec = item_flat.reshape(B, D_ITEM)
    pb = jnp.concatenate(
        [Price.astype(jnp.float32)[:, None],
         IsBestSeller.astype(jnp.float32)[:, None]], axis=1)
    return _mlp(
        item_vec, gh_vec, gn_vec, pb,
        W1[:D_ITEM], W1[D_ITEM:D_ITEM + D_GROUP],
        W1[D_ITEM + D_GROUP:D_ITEM + 2 * D_GROUP], W1[D_ITEM + 2 * D_GROUP:],
        b1[None, :], W2, b2[None, :], Wp, bp[None, :])


# speedup vs baseline: 6.2591x; 6.2591x over previous
import jax
import jax.numpy as jnp
from jax import lax
from jax.experimental import pallas as pl
from jax.experimental.pallas import tpu as pltpu
from jax.experimental.pallas import tpu_sc as plsc

B = 16384
ITEM_VOCAB = 1000000
D_ITEM = 32
D_GROUP = 16
H1, H2 = 128, 64

NC, NS = 2, 16
NW = NC * NS
BPW = B // NW
CH = 256
G = 16

BLK = 2048


def _gather_body(pc_hbm, ghl_hbm, gnl_hbm, tbl, gh128, gn128,
                 item_out, ghl_out, gnl_out,
                 pc_v, lidx_v, slab_v, rows_v, lines_v,
                 sem_g, sem_i):
    wid = lax.axis_index("s") * NC + lax.axis_index("c")
    base = wid * BPW

    for idx_hbm, src, out_hbm in ((ghl_hbm, gh128, ghl_out),
                                  (gnl_hbm, gn128, gnl_out)):
        for ch in range(BPW // CH):
            off = base + ch * CH
            pltpu.sync_copy(idx_hbm.at[pl.ds(off, CH)], lidx_v)
            pltpu.async_copy(src.at[lidx_v], lines_v, sem_g).wait()
            pltpu.sync_copy(lines_v, out_hbm.at[pl.ds(off, CH)])

    pltpu.sync_copy(pc_hbm.at[pl.ds(base, BPW)], pc_v)

    def group(g, carry):
        gbase = g * G
        pc16 = pc_v[pl.ds(gbase, G)]
        for k in range(G):
            rb = pc16[k] >> 3
            r8 = pl.multiple_of(rb * 8, 8)
            pltpu.async_copy(tbl.at[pl.ds(r8, 8)], slab_v.at[k], sem_i)
        for k in range(G):
            pltpu.make_async_copy(tbl.at[pl.ds(0, 8)], slab_v.at[0],
                                  sem_i).wait()
        for k in range(G):
            rem = jnp.bitwise_and(pc16[k], 7)
            for hh in range(2):
                rows_v[gbase + k, pl.ds(hh * 16, 16)] = (
                    slab_v[k, rem, pl.ds(hh * 16, 16)])
        return carry

    lax.fori_loop(0, BPW // G, group, 0)
    pltpu.sync_copy(rows_v, item_out.at[pl.ds(base, BPW)])


def _sc_gather():
    return pl.kernel(
        _gather_body,
        out_type=[
            jax.ShapeDtypeStruct((B, D_ITEM), jnp.float32),
            jax.ShapeDtypeStruct((B, 128), jnp.float32),
            jax.ShapeDtypeStruct((B, 128), jnp.float32),
        ],
        mesh=plsc.VectorSubcoreMesh(
            core_axis_name="c", subcore_axis_name="s",
            num_cores=NC, num_subcores=NS),
        compiler_params=pltpu.CompilerParams(use_tc_tiling_on_sc=True),
        scratch_types=[
            pltpu.VMEM((BPW,), jnp.int32),
            pltpu.VMEM((CH,), jnp.int32),
            pltpu.VMEM((G, 8, D_ITEM), jnp.float32),
            pltpu.VMEM((BPW, D_ITEM), jnp.float32),
            pltpu.VMEM((CH, 128), jnp.float32),
            pltpu.SemaphoreType.DMA,
            pltpu.SemaphoreType.DMA,
        ],
    )


def _mlp_body(item_ref, ghl_ref, gnl_ref, pbr_ref,
              w1a_ref, w1b8_ref, w1c8_ref, w1pq_ref, b1_ref,
              w2_ref, b2_ref, wp_ref, bp_ref, out_ref):
    h = jnp.dot(item_ref[...], w1a_ref[...], preferred_element_type=jnp.float32)
    seg8 = (lax.broadcasted_iota(jnp.int32, (BLK, 128), 1) >> 4).astype(
        jnp.float32)
    gh = jnp.where(seg8 == pbr_ref[:, 2:3], ghl_ref[...], 0.0)
    gn = jnp.where(seg8 == pbr_ref[:, 3:4], gnl_ref[...], 0.0)
    h += jnp.dot(gh, w1b8_ref[...], preferred_element_type=jnp.float32)
    h += jnp.dot(gn, w1c8_ref[...], preferred_element_type=jnp.float32)
    h += jnp.dot(pbr_ref[:, 0:2], w1pq_ref[...],
                 preferred_element_type=jnp.float32)
    h = jnp.maximum(h + b1_ref[...], 0.0)
    h = jnp.maximum(
        jnp.dot(h, w2_ref[...], preferred_element_type=jnp.float32) + b2_ref[...],
        0.0)
    out_ref[...] = (
        jnp.dot(h, wp_ref[...], preferred_element_type=jnp.float32) + bp_ref[...])


def _mlp(item_vec, ghl, gnl, pbr, W1a, W1b8, W1c8, W1pq, b1, W2, b2, Wp, bp,
         interpret=False):
    row = lambda i: (i, 0)
    rep = lambda i: (0, 0)
    return pl.pallas_call(
        _mlp_body,
        grid=(B // BLK,),
        in_specs=[
            pl.BlockSpec((BLK, D_ITEM), row),
            pl.BlockSpec((BLK, 128), row),
            pl.BlockSpec((BLK, 128), row),
            pl.BlockSpec((BLK, 4), row),
            pl.BlockSpec((D_ITEM, H1), rep),
            pl.BlockSpec((128, H1), rep),
            pl.BlockSpec((128, H1), rep),
            pl.BlockSpec((2, H1), rep),
            pl.BlockSpec((1, H1), rep),
            pl.BlockSpec((H1, H2), rep),
            pl.BlockSpec((1, H2), rep),
            pl.BlockSpec((H2, H2), rep),
            pl.BlockSpec((1, H2), rep),
        ],
        out_specs=pl.BlockSpec((BLK, H2), row),
        out_shape=jax.ShapeDtypeStruct((B, H2), jnp.float32),
        interpret=interpret,
    )(item_vec, ghl, gnl, pbr, W1a, W1b8, W1c8, W1pq, b1, W2, b2, Wp, bp)


def kernel(ProductCode, ProductGroupHeader, ProductGroupName, Price, IsBestSeller,
           item_table, gh_table, gn_table, W1, b1, W2, b2, Wp, bp):
    gh128 = gh_table.reshape(125, 128)
    gn128 = gn_table.reshape(125, 128)
    item_vec, ghl, gnl = _sc_gather()(
        ProductCode, ProductGroupHeader >> 3, ProductGroupName >> 3,
        item_table, gh128, gn128)
    pbr = jnp.concatenate(
        [Price.astype(jnp.float32)[:, None],
         IsBestSeller.astype(jnp.float32)[:, None],
         (ProductGroupHeader & 7).astype(jnp.float32)[:, None],
         (ProductGroupName & 7).astype(jnp.float32)[:, None]], axis=1)
    W1b = W1[D_ITEM:D_ITEM + D_GROUP]
    W1c = W1[D_ITEM + D_GROUP:D_ITEM + 2 * D_GROUP]
    W1b8 = jnp.concatenate([W1b] * 8, axis=0)
    W1c8 = jnp.concatenate([W1c] * 8, axis=0)
    return _mlp(
        item_vec, ghl, gnl, pbr,
        W1[:D_ITEM], W1b8, W1c8, W1[D_ITEM + 2 * D_GROUP:],
        b1[None, :], W2, b2[None, :], Wp, bp[None, :])


# speedup vs baseline: 6.4911x; 1.0371x over previous
import jax
import jax.numpy as jnp
from jax import lax
from jax.experimental import pallas as pl
from jax.experimental.pallas import tpu as pltpu
from jax.experimental.pallas import tpu_sc as plsc

B = 16384
ITEM_VOCAB = 1000000
D_ITEM = 32
D_GROUP = 16
H1, H2 = 128, 64

NC, NS = 2, 16
NW = NC * NS
BPW = B // NW
CH = 256
G = 16
NG = BPW // G

BLK = 2048

def _sc_params():
    return dict(
        mesh=plsc.VectorSubcoreMesh(
            core_axis_name="c", subcore_axis_name="s",
            num_cores=NC, num_subcores=NS),
        compiler_params=pltpu.CompilerParams(use_tc_tiling_on_sc=True),
    )


def _group_body(ghl_hbm, gnl_hbm, gh128, gn128,
                ghl_out, gnl_out, lidx_v, lines_v, sem_g):
    wid = lax.axis_index("s") * NC + lax.axis_index("c")
    base = wid * BPW
    for idx_hbm, src, out_hbm in ((ghl_hbm, gh128, ghl_out),
                                  (gnl_hbm, gn128, gnl_out)):
        for ch in range(BPW // CH):
            off = base + ch * CH
            pltpu.sync_copy(idx_hbm.at[pl.ds(off, CH)], lidx_v)
            pltpu.async_copy(src.at[lidx_v], lines_v, sem_g).wait()
            pltpu.sync_copy(lines_v, out_hbm.at[pl.ds(off, CH)])


def _sc_groups():
    return pl.kernel(
        _group_body,
        out_type=[
            jax.ShapeDtypeStruct((B, 128), jnp.float32),
            jax.ShapeDtypeStruct((B, 128), jnp.float32),
        ],
        scratch_types=[
            pltpu.VMEM((CH,), jnp.int32),
            pltpu.VMEM((CH, 128), jnp.float32),
            pltpu.SemaphoreType.DMA,
        ],
        **_sc_params(),
    )


def _item_body(pc_hbm, tbl, item_out, pc_v, slab0, slab1, rows_v,
               sem0, sem1):
    wid = lax.axis_index("s") * NC + lax.axis_index("c")
    base = wid * BPW
    pltpu.sync_copy(pc_hbm.at[pl.ds(base, BPW)], pc_v)

    def fire(gbase, slab, sem):
        pc16 = pc_v[pl.ds(gbase, G)]
        for k in range(G):
            rb = pc16[k] >> 3
            r8 = pl.multiple_of(rb * 8, 8)
            pltpu.async_copy(tbl.at[pl.ds(r8, 8)], slab.at[k], sem)
        return pc16

    def drain(slab, sem):
        for _ in range(G):
            pltpu.make_async_copy(tbl.at[pl.ds(0, 8)], slab.at[0], sem).wait()

    def extract(gbase, pc16, slab):
        for k in range(G):
            rem = jnp.bitwise_and(pc16[k], 7)
            for hh in range(2):
                rows_v[gbase + k, pl.ds(hh * 16, 16)] = (
                    slab[k, rem, pl.ds(hh * 16, 16)])

    fire(0, slab0, sem0)

    def pair(gp, carry):
        g0 = gp * 2 * G
        g1 = g0 + G
        pc0 = pc_v[pl.ds(g0, G)]
        fire(g1, slab1, sem1)
        drain(slab0, sem0)
        extract(g0, pc0, slab0)

        @pl.when(gp + 1 < NG // 2)
        def _():
            fire(g1 + G, slab0, sem0)

        pc1 = pc_v[pl.ds(g1, G)]
        drain(slab1, sem1)
        extract(g1, pc1, slab1)
        return carry

    lax.fori_loop(0, NG // 2, pair, 0)
    pltpu.sync_copy(rows_v, item_out.at[pl.ds(base, BPW)])


def _sc_item():
    return pl.kernel(
        _item_body,
        out_type=jax.ShapeDtypeStruct((B, D_ITEM), jnp.float32),
        scratch_types=[
            pltpu.VMEM((BPW,), jnp.int32),
            pltpu.VMEM((G, 8, D_ITEM), jnp.float32),
            pltpu.VMEM((G, 8, D_ITEM), jnp.float32),
            pltpu.VMEM((BPW, D_ITEM), jnp.float32),
            pltpu.SemaphoreType.DMA,
            pltpu.SemaphoreType.DMA,
        ],
        **_sc_params(),
    )


def _mlp_body(item_ref, ghl_ref, gnl_ref, pbr_ref,
              w1a_ref, w1b8_ref, w1c8_ref, w1pq_ref, b1_ref,
              w2_ref, b2_ref, wp_ref, bp_ref, out_ref):
    h = jnp.dot(item_ref[...], w1a_ref[...], preferred_element_type=jnp.float32)
    seg8 = (lax.broadcasted_iota(jnp.int32, (BLK, 128), 1) >> 4).astype(
        jnp.float32)
    gh = jnp.where(seg8 == pbr_ref[:, 2:3], ghl_ref[...], 0.0)
    gn = jnp.where(seg8 == pbr_ref[:, 3:4], gnl_ref[...], 0.0)
    h += jnp.dot(gh, w1b8_ref[...], preferred_element_type=jnp.float32)
    h += jnp.dot(gn, w1c8_ref[...], preferred_element_type=jnp.float32)
    h += jnp.dot(pbr_ref[:, 0:2], w1pq_ref[...],
                 preferred_element_type=jnp.float32)
    h = jnp.maximum(h + b1_ref[...], 0.0)
    h = jnp.maximum(
        jnp.dot(h, w2_ref[...], preferred_element_type=jnp.float32) + b2_ref[...],
        0.0)
    out_ref[...] = (
        jnp.dot(h, wp_ref[...], preferred_element_type=jnp.float32) + bp_ref[...])


def _mlp(item_vec, ghl, gnl, pbr, W1a, W1b8, W1c8, W1pq, b1, W2, b2, Wp, bp,
         interpret=False):
    row = lambda i: (i, 0)
    rep = lambda i: (0, 0)
    return pl.pallas_call(
        _mlp_body,
        grid=(B // BLK,),
        in_specs=[
            pl.BlockSpec((BLK, D_ITEM), row),
            pl.BlockSpec((BLK, 128), row),
            pl.BlockSpec((BLK, 128), row),
            pl.BlockSpec((BLK, 4), row),
            pl.BlockSpec((D_ITEM, H1), rep),
            pl.BlockSpec((128, H1), rep),
            pl.BlockSpec((128, H1), rep),
            pl.BlockSpec((2, H1), rep),
            pl.BlockSpec((1, H1), rep),
            pl.BlockSpec((H1, H2), rep),
            pl.BlockSpec((1, H2), rep),
            pl.BlockSpec((H2, H2), rep),
            pl.BlockSpec((1, H2), rep),
        ],
        out_specs=pl.BlockSpec((BLK, H2), row),
        out_shape=jax.ShapeDtypeStruct((B, H2), jnp.float32),
        interpret=interpret,
    )(item_vec, ghl, gnl, pbr, W1a, W1b8, W1c8, W1pq, b1, W2, b2, Wp, bp)


def kernel(ProductCode, ProductGroupHeader, ProductGroupName, Price, IsBestSeller,
           item_table, gh_table, gn_table, W1, b1, W2, b2, Wp, bp):
    gh128 = gh_table.reshape(125, 128)
    gn128 = gn_table.reshape(125, 128)
    ghl, gnl = _sc_groups()(
        ProductGroupHeader >> 3, ProductGroupName >> 3, gh128, gn128)
    item_vec = _sc_item()(ProductCode, item_table)
    pbr = jnp.concatenate(
        [Price.astype(jnp.float32)[:, None],
         IsBestSeller.astype(jnp.float32)[:, None],
         (ProductGroupHeader & 7).astype(jnp.float32)[:, None],
         (ProductGroupName & 7).astype(jnp.float32)[:, None]], axis=1)
    W1b = W1[D_ITEM:D_ITEM + D_GROUP]
    W1c = W1[D_ITEM + D_GROUP:D_ITEM + 2 * D_GROUP]
    W1b8 = jnp.concatenate([W1b] * 8, axis=0)
    W1c8 = jnp.concatenate([W1c] * 8, axis=0)
    return _mlp(
        item_vec, ghl, gnl, pbr,
        W1[:D_ITEM], W1b8, W1c8, W1[D_ITEM + 2 * D_GROUP:],
        b1[None, :], W2, b2[None, :], Wp, bp[None, :])


# speedup vs baseline: 9.2527x; 1.4254x over previous
import jax
import jax.numpy as jnp
from jax import lax
from jax.experimental import pallas as pl
from jax.experimental.pallas import tpu as pltpu
from jax.experimental.pallas import tpu_sc as plsc

B = 16384
ITEM_VOCAB = 1000000
D_ITEM = 32
D_GROUP = 16
H1, H2 = 128, 64

NC, NS = 2, 16
NW = NC * NS
BPW = B // NW
CH = 256
G = 16
NG = BPW // G

BLK = 2048

def _sc_params():
    return dict(
        mesh=plsc.VectorSubcoreMesh(
            core_axis_name="c", subcore_axis_name="s",
            num_cores=NC, num_subcores=NS),
        compiler_params=pltpu.CompilerParams(use_tc_tiling_on_sc=True),
    )


def _group_body(ghl_hbm, gnl_hbm, gh128, gn128,
                ghl_out, gnl_out, lidx_v, lines_v, sem_g):
    wid = lax.axis_index("s") * NC + lax.axis_index("c")
    base = wid * BPW
    for idx_hbm, src, out_hbm in ((ghl_hbm, gh128, ghl_out),
                                  (gnl_hbm, gn128, gnl_out)):
        for ch in range(BPW // CH):
            off = base + ch * CH
            pltpu.sync_copy(idx_hbm.at[pl.ds(off, CH)], lidx_v)
            pltpu.async_copy(src.at[lidx_v], lines_v, sem_g).wait()
            pltpu.sync_copy(lines_v, out_hbm.at[pl.ds(off, CH)])


def _sc_groups():
    return pl.kernel(
        _group_body,
        out_type=[
            jax.ShapeDtypeStruct((B, 128), jnp.float32),
            jax.ShapeDtypeStruct((B, 128), jnp.float32),
        ],
        scratch_types=[
            pltpu.VMEM((CH,), jnp.int32),
            pltpu.VMEM((CH, 128), jnp.float32),
            pltpu.SemaphoreType.DMA,
        ],
        **_sc_params(),
    )


def _item_body(pc_hbm, tbl, item_out, pc_v, slab0, slab1, rows_v,
               sem0, sem1):
    wid = lax.axis_index("s") * NC + lax.axis_index("c")
    base = wid * BPW
    pltpu.sync_copy(pc_hbm.at[pl.ds(base, BPW)], pc_v)

    def fire(gbase, slab, sem):
        pc16 = pc_v[pl.ds(gbase, G)]
        for k in range(G):
            rb = pc16[k] >> 3
            pltpu.async_copy(tbl.at[rb], slab.at[k], sem)
        return pc16

    def drain(slab, sem):
        for _ in range(G):
            pltpu.make_async_copy(tbl.at[0], slab.at[0], sem).wait()

    def extract(gbase, pc16, slab):
        for k in range(G):
            rem = jnp.bitwise_and(pc16[k], 7)
            for hh in range(2):
                rows_v[gbase + k, pl.ds(hh * 16, 16)] = (
                    slab[k, rem, pl.ds(hh * 16, 16)])

    fire(0, slab0, sem0)

    def pair(gp, carry):
        g0 = gp * 2 * G
        g1 = g0 + G
        pc0 = pc_v[pl.ds(g0, G)]
        fire(g1, slab1, sem1)
        drain(slab0, sem0)
        extract(g0, pc0, slab0)

        @pl.when(gp + 1 < NG // 2)
        def _():
            fire(g1 + G, slab0, sem0)

        pc1 = pc_v[pl.ds(g1, G)]
        drain(slab1, sem1)
        extract(g1, pc1, slab1)
        return carry

    lax.fori_loop(0, NG // 2, pair, 0)
    pltpu.sync_copy(rows_v, item_out.at[pl.ds(base, BPW)])


def _sc_item():
    return pl.kernel(
        _item_body,
        out_type=jax.ShapeDtypeStruct((B, D_ITEM), jnp.float32),
        scratch_types=[
            pltpu.VMEM((BPW,), jnp.int32),
            pltpu.VMEM((G, 8, D_ITEM), jnp.float32),
            pltpu.VMEM((G, 8, D_ITEM), jnp.float32),
            pltpu.VMEM((BPW, D_ITEM), jnp.float32),
            pltpu.SemaphoreType.DMA,
            pltpu.SemaphoreType.DMA,
        ],
        **_sc_params(),
    )


def _mlp_body(item_ref, ghl_ref, gnl_ref, pbr_ref,
              w1a_ref, w1b8_ref, w1c8_ref, w1pq_ref, b1_ref,
              w2_ref, b2_ref, wp_ref, bp_ref, out_ref):
    h = jnp.dot(item_ref[...], w1a_ref[...], preferred_element_type=jnp.float32)
    seg8 = (lax.broadcasted_iota(jnp.int32, (BLK, 128), 1) >> 4).astype(
        jnp.float32)
    gh = jnp.where(seg8 == pbr_ref[:, 2:3], ghl_ref[...], 0.0)
    gn = jnp.where(seg8 == pbr_ref[:, 3:4], gnl_ref[...], 0.0)
    h += jnp.dot(gh, w1b8_ref[...], preferred_element_type=jnp.float32)
    h += jnp.dot(gn, w1c8_ref[...], preferred_element_type=jnp.float32)
    h += jnp.dot(pbr_ref[:, 0:2], w1pq_ref[...],
                 preferred_element_type=jnp.float32)
    h = jnp.maximum(h + b1_ref[...], 0.0)
    h = jnp.maximum(
        jnp.dot(h, w2_ref[...], preferred_element_type=jnp.float32) + b2_ref[...],
        0.0)
    out_ref[...] = (
        jnp.dot(h, wp_ref[...], preferred_element_type=jnp.float32) + bp_ref[...])


def _mlp(item_vec, ghl, gnl, pbr, W1a, W1b8, W1c8, W1pq, b1, W2, b2, Wp, bp,
         interpret=False):
    row = lambda i: (i, 0)
    rep = lambda i: (0, 0)
    return pl.pallas_call(
        _mlp_body,
        grid=(B // BLK,),
        in_specs=[
            pl.BlockSpec((BLK, D_ITEM), row),
            pl.BlockSpec((BLK, 128), row),
            pl.BlockSpec((BLK, 128), row),
            pl.BlockSpec((BLK, 4), row),
            pl.BlockSpec((D_ITEM, H1), rep),
            pl.BlockSpec((128, H1), rep),
            pl.BlockSpec((128, H1), rep),
            pl.BlockSpec((2, H1), rep),
            pl.BlockSpec((1, H1), rep),
            pl.BlockSpec((H1, H2), rep),
            pl.BlockSpec((1, H2), rep),
            pl.BlockSpec((H2, H2), rep),
            pl.BlockSpec((1, H2), rep),
        ],
        out_specs=pl.BlockSpec((BLK, H2), row),
        out_shape=jax.ShapeDtypeStruct((B, H2), jnp.float32),
        interpret=interpret,
    )(item_vec, ghl, gnl, pbr, W1a, W1b8, W1c8, W1pq, b1, W2, b2, Wp, bp)


def kernel(ProductCode, ProductGroupHeader, ProductGroupName, Price, IsBestSeller,
           item_table, gh_table, gn_table, W1, b1, W2, b2, Wp, bp):
    gh128 = gh_table.reshape(125, 128)
    gn128 = gn_table.reshape(125, 128)
    ghl, gnl = _sc_groups()(
        ProductGroupHeader >> 3, ProductGroupName >> 3, gh128, gn128)
    item_vec = _sc_item()(ProductCode, item_table.reshape(125000, 8, D_ITEM))
    pbr = jnp.concatenate(
        [Price.astype(jnp.float32)[:, None],
         IsBestSeller.astype(jnp.float32)[:, None],
         (ProductGroupHeader & 7).astype(jnp.float32)[:, None],
         (ProductGroupName & 7).astype(jnp.float32)[:, None]], axis=1)
    W1b = W1[D_ITEM:D_ITEM + D_GROUP]
    W1c = W1[D_ITEM + D_GROUP:D_ITEM + 2 * D_GROUP]
    W1b8 = jnp.concatenate([W1b] * 8, axis=0)
    W1c8 = jnp.concatenate([W1c] * 8, axis=0)
    return _mlp(
        item_vec, ghl, gnl, pbr,
        W1[:D_ITEM], W1b8, W1c8, W1[D_ITEM + 2 * D_GROUP:],
        b1[None, :], W2, b2[None, :], Wp, bp[None, :])


# speedup vs baseline: 9.3079x; 1.0060x over previous
import jax
import jax.numpy as jnp
from jax import lax
from jax.experimental import pallas as pl
from jax.experimental.pallas import tpu as pltpu
from jax.experimental.pallas import tpu_sc as plsc

B = 16384
ITEM_VOCAB = 1000000
D_ITEM = 32
D_GROUP = 16
H1, H2 = 128, 64

NC, NS = 2, 16
NW = NC * NS
BPW = B // NW
CH = 256
G = 16
NG = BPW // G

BLK = 2048


def _sc_params():
    return dict(
        mesh=plsc.VectorSubcoreMesh(
            core_axis_name="c", subcore_axis_name="s",
            num_cores=NC, num_subcores=NS),
        compiler_params=pltpu.CompilerParams(use_tc_tiling_on_sc=True),
    )


def _group_body(ghl_hbm, gnl_hbm, gh128, gn128,
                ghl_out, gnl_out, lidx0, lidx1, lines0, lines1,
                sem0, sem1):
    wid = lax.axis_index("s") * NC + lax.axis_index("c")
    base = wid * BPW
    chunks = [(idx_hbm, src, out_hbm, base + ch * CH)
              for idx_hbm, src, out_hbm in ((ghl_hbm, gh128, ghl_out),
                                            (gnl_hbm, gn128, gnl_out))
              for ch in range(BPW // CH)]
    bufs = ((lidx0, lines0, sem0), (lidx1, lines1, sem1))
    prev = None
    for t, (idx_hbm, src, out_hbm, off) in enumerate(chunks):
        lidx, lines, sem = bufs[t % 2]
        pltpu.sync_copy(idx_hbm.at[pl.ds(off, CH)], lidx)
        c = pltpu.async_copy(src.at[lidx], lines, sem)
        if prev is not None:
            pc, plines, pout, poff = prev
            pc.wait()
            pltpu.sync_copy(plines, pout.at[pl.ds(poff, CH)])
        prev = (c, lines, out_hbm, off)
    pc, plines, pout, poff = prev
    pc.wait()
    pltpu.sync_copy(plines, pout.at[pl.ds(poff, CH)])


def _sc_groups():
    return pl.kernel(
        _group_body,
        out_type=[
            jax.ShapeDtypeStruct((B, 128), jnp.float32),
            jax.ShapeDtypeStruct((B, 128), jnp.float32),
        ],
        scratch_types=[
            pltpu.VMEM((CH,), jnp.int32),
            pltpu.VMEM((CH,), jnp.int32),
            pltpu.VMEM((CH, 128), jnp.float32),
            pltpu.VMEM((CH, 128), jnp.float32),
            pltpu.SemaphoreType.DMA,
            pltpu.SemaphoreType.DMA,
        ],
        **_sc_params(),
    )


def _item_body(pc_hbm, tbl, item_out, pc_v, slab0, slab1, rows_v,
               sem0, sem1):
    wid = lax.axis_index("s") * NC + lax.axis_index("c")
    base = wid * BPW
    pltpu.sync_copy(pc_hbm.at[pl.ds(base, BPW)], pc_v)

    def fire(gbase, slab, sem):
        pc16 = pc_v[pl.ds(gbase, G)]
        for k in range(G):
            rb = pc16[k] >> 3
            pltpu.async_copy(tbl.at[rb], slab.at[k], sem)
        return pc16

    def drain(slab, sem):
        for _ in range(G):
            pltpu.make_async_copy(tbl.at[0], slab.at[0], sem).wait()

    def extract(gbase, pc16, slab):
        for k in range(G):
            rem = jnp.bitwise_and(pc16[k], 7)
            for hh in range(2):
                rows_v[gbase + k, pl.ds(hh * 16, 16)] = (
                    slab[k, rem, pl.ds(hh * 16, 16)])

    fire(0, slab0, sem0)

    def pair(gp, carry):
        g0 = gp * 2 * G
        g1 = g0 + G
        pc0 = pc_v[pl.ds(g0, G)]
        fire(g1, slab1, sem1)
        drain(slab0, sem0)
        extract(g0, pc0, slab0)

        @pl.when(gp + 1 < NG // 2)
        def _():
            fire(g1 + G, slab0, sem0)

        pc1 = pc_v[pl.ds(g1, G)]
        drain(slab1, sem1)
        extract(g1, pc1, slab1)
        return carry

    lax.fori_loop(0, NG // 2, pair, 0)
    pltpu.sync_copy(rows_v, item_out.at[pl.ds(base, BPW)])


def _sc_item():
    return pl.kernel(
        _item_body,
        out_type=jax.ShapeDtypeStruct((B, D_ITEM), jnp.float32),
        scratch_types=[
            pltpu.VMEM((BPW,), jnp.int32),
            pltpu.VMEM((G, 8, D_ITEM), jnp.float32),
            pltpu.VMEM((G, 8, D_ITEM), jnp.float32),
            pltpu.VMEM((BPW, D_ITEM), jnp.float32),
            pltpu.SemaphoreType.DMA,
            pltpu.SemaphoreType.DMA,
        ],
        **_sc_params(),
    )


def _mlp_body(item_ref, ghl_ref, gnl_ref, pbr_ref,
              w1a_ref, w1b8_ref, w1c8_ref, w1pq_ref, b1_ref,
              w2_ref, b2_ref, wp_ref, bp_ref, out_ref):
    h = jnp.dot(item_ref[...], w1a_ref[...], preferred_element_type=jnp.float32)
    seg8 = (lax.broadcasted_iota(jnp.int32, (BLK, 128), 1) >> 4).astype(
        jnp.float32)
    gh = jnp.where(seg8 == pbr_ref[:, 2:3], ghl_ref[...], 0.0)
    gn = jnp.where(seg8 == pbr_ref[:, 3:4], gnl_ref[...], 0.0)
    h += jnp.dot(gh, w1b8_ref[...], preferred_element_type=jnp.float32)
    h += jnp.dot(gn, w1c8_ref[...], preferred_element_type=jnp.float32)
    h += jnp.dot(pbr_ref[:, 0:2], w1pq_ref[...],
                 preferred_element_type=jnp.float32)
    h = jnp.maximum(h + b1_ref[...], 0.0)
    h = jnp.maximum(
        jnp.dot(h, w2_ref[...], preferred_element_type=jnp.float32) + b2_ref[...],
        0.0)
    out_ref[...] = (
        jnp.dot(h, wp_ref[...], preferred_element_type=jnp.float32) + bp_ref[...])


def _mlp(item_vec, ghl, gnl, pbr, W1a, W1b8, W1c8, W1pq, b1, W2, b2, Wp, bp,
         interpret=False):
    row = lambda i: (i, 0)
    rep = lambda i: (0, 0)
    return pl.pallas_call(
        _mlp_body,
        grid=(B // BLK,),
        in_specs=[
            pl.BlockSpec((BLK, D_ITEM), row),
            pl.BlockSpec((BLK, 128), row),
            pl.BlockSpec((BLK, 128), row),
            pl.BlockSpec((BLK, 4), row),
            pl.BlockSpec((D_ITEM, H1), rep),
            pl.BlockSpec((128, H1), rep),
            pl.BlockSpec((128, H1), rep),
            pl.BlockSpec((2, H1), rep),
            pl.BlockSpec((1, H1), rep),
            pl.BlockSpec((H1, H2), rep),
            pl.BlockSpec((1, H2), rep),
            pl.BlockSpec((H2, H2), rep),
            pl.BlockSpec((1, H2), rep),
        ],
        out_specs=pl.BlockSpec((BLK, H2), row),
        out_shape=jax.ShapeDtypeStruct((B, H2), jnp.float32),
        interpret=interpret,
    )(item_vec, ghl, gnl, pbr, W1a, W1b8, W1c8, W1pq, b1, W2, b2, Wp, bp)


def kernel(ProductCode, ProductGroupHeader, ProductGroupName, Price, IsBestSeller,
           item_table, gh_table, gn_table, W1, b1, W2, b2, Wp, bp):
    gh128 = gh_table.reshape(125, 128)
    gn128 = gn_table.reshape(125, 128)
    ghl, gnl = _sc_groups()(
        ProductGroupHeader >> 3, ProductGroupName >> 3, gh128, gn128)
    item_vec = _sc_item()(ProductCode, item_table.reshape(125000, 8, D_ITEM))
    pbr = jnp.concatenate(
        [Price.astype(jnp.float32)[:, None],
         IsBestSeller.astype(jnp.float32)[:, None],
         (ProductGroupHeader & 7).astype(jnp.float32)[:, None],
         (ProductGroupName & 7).astype(jnp.float32)[:, None]], axis=1)
    W1b = W1[D_ITEM:D_ITEM + D_GROUP]
    W1c = W1[D_ITEM + D_GROUP:D_ITEM + 2 * D_GROUP]
    W1b8 = jnp.concatenate([W1b] * 8, axis=0)
    W1c8 = jnp.concatenate([W1c] * 8, axis=0)
    return _mlp(
        item_vec, ghl, gnl, pbr,
        W1[:D_ITEM], W1b8, W1c8, W1[D_ITEM + 2 * D_GROUP:],
        b1[None, :], W2, b2[None, :], Wp, bp[None, :])


# speedup vs baseline: 9.5418x; 1.0251x over previous
import jax
import jax.numpy as jnp
from jax import lax
from jax.experimental import pallas as pl
from jax.experimental.pallas import tpu as pltpu
from jax.experimental.pallas import tpu_sc as plsc

B = 16384
ITEM_VOCAB = 1000000
D_ITEM = 32
D_GROUP = 16
H1, H2 = 128, 64

NC, NS = 2, 16
NW = NC * NS
BPW = B // NW
CH = 256
G = 16
NG = BPW // G

BLK = 2048


def _sc_params():
    return dict(
        mesh=plsc.VectorSubcoreMesh(
            core_axis_name="c", subcore_axis_name="s",
            num_cores=NC, num_subcores=NS),
        compiler_params=pltpu.CompilerParams(use_tc_tiling_on_sc=True),
    )


def _group_body(ghl_hbm, gnl_hbm, gh128, gn128,
                ghl_out, gnl_out, lidx0, lidx1, lines0, lines1,
                sem0, sem1):
    wid = lax.axis_index("s") * NC + lax.axis_index("c")
    base = wid * BPW
    chunks = [(idx_hbm, src, out_hbm, base + ch * CH)
              for idx_hbm, src, out_hbm in ((ghl_hbm, gh128, ghl_out),
                                            (gnl_hbm, gn128, gnl_out))
              for ch in range(BPW // CH)]
    bufs = ((lidx0, lines0, sem0), (lidx1, lines1, sem1))
    prev = None
    for t, (idx_hbm, src, out_hbm, off) in enumerate(chunks):
        lidx, lines, sem = bufs[t % 2]
        pltpu.sync_copy(idx_hbm.at[pl.ds(off, CH)], lidx)
        c = pltpu.async_copy(src.at[lidx], lines, sem)
        if prev is not None:
            pc, plines, pout, poff = prev
            pc.wait()
            pltpu.sync_copy(plines, pout.at[pl.ds(poff, CH)])
        prev = (c, lines, out_hbm, off)
    pc, plines, pout, poff = prev
    pc.wait()
    pltpu.sync_copy(plines, pout.at[pl.ds(poff, CH)])


def _sc_groups():
    return pl.kernel(
        _group_body,
        out_type=[
            jax.ShapeDtypeStruct((B, 128), jnp.float32),
            jax.ShapeDtypeStruct((B, 128), jnp.float32),
        ],
        scratch_types=[
            pltpu.VMEM((CH,), jnp.int32),
            pltpu.VMEM((CH,), jnp.int32),
            pltpu.VMEM((CH, 128), jnp.float32),
            pltpu.VMEM((CH, 128), jnp.float32),
            pltpu.SemaphoreType.DMA,
            pltpu.SemaphoreType.DMA,
        ],
        **_sc_params(),
    )


def _item_body(pc_hbm, tbl, item_out, pc_v, slab0, slab1, rows_v,
               sem0, sem1):
    wid = lax.axis_index("s") * NC + lax.axis_index("c")
    base = wid * BPW
    pltpu.sync_copy(pc_hbm.at[pl.ds(base, BPW)], pc_v)

    def fire(gbase, slab, sem):
        pc16 = pc_v[pl.ds(gbase, G)]
        for k in range(G):
            rb = pc16[k] >> 3
            pltpu.async_copy(tbl.at[rb], slab.at[k], sem)
        return pc16

    def drain(slab, sem):
        for _ in range(G):
            pltpu.make_async_copy(tbl.at[0], slab.at[0], sem).wait()

    def extract(gbase, pc16, slab):
        for k in range(G):
            rem = jnp.bitwise_and(pc16[k], 7)
            for hh in range(2):
                rows_v[gbase + k, pl.ds(hh * 16, 16)] = (
                    slab[k, rem, pl.ds(hh * 16, 16)])

    fire(0, slab0, sem0)

    def pair(gp, carry):
        g0 = gp * 2 * G
        g1 = g0 + G
        pc0 = pc_v[pl.ds(g0, G)]
        fire(g1, slab1, sem1)
        drain(slab0, sem0)
        extract(g0, pc0, slab0)

        @pl.when(gp + 1 < NG // 2)
        def _():
            fire(g1 + G, slab0, sem0)

        pc1 = pc_v[pl.ds(g1, G)]
        drain(slab1, sem1)
        extract(g1, pc1, slab1)
        return carry

    lax.fori_loop(0, NG // 2, pair, 0)
    pltpu.sync_copy(rows_v, item_out.at[pl.ds(base, BPW)])


def _sc_item():
    return pl.kernel(
        _item_body,
        out_type=jax.ShapeDtypeStruct((B, D_ITEM), jnp.float32),
        scratch_types=[
            pltpu.VMEM((BPW,), jnp.int32),
            pltpu.VMEM((G, 8, D_ITEM), jnp.float32),
            pltpu.VMEM((G, 8, D_ITEM), jnp.float32),
            pltpu.VMEM((BPW, D_ITEM), jnp.float32),
            pltpu.SemaphoreType.DMA,
            pltpu.SemaphoreType.DMA,
        ],
        **_sc_params(),
    )


def _mlp_body(item_ref, ghl_ref, gnl_ref, pbr_ref,
              w1a_ref, w1b8_ref, w1c8_ref, w1pq_ref, b1_ref,
              w2_ref, b2_ref, wp_ref, bp_ref, out_ref):
    h = jnp.dot(item_ref[...], w1a_ref[...], preferred_element_type=jnp.float32)
    seg8 = (lax.broadcasted_iota(jnp.int32, (BLK, 128), 1) >> 4).astype(
        jnp.float32)
    gh = jnp.where(seg8 == pbr_ref[:, 2:3], ghl_ref[...], 0.0)
    gn = jnp.where(seg8 == pbr_ref[:, 3:4], gnl_ref[...], 0.0)
    h += jnp.dot(gh, w1b8_ref[...], preferred_element_type=jnp.float32)
    h += jnp.dot(gn, w1c8_ref[...], preferred_element_type=jnp.float32)
    h += jnp.dot(pbr_ref[:, 0:2], w1pq_ref[...],
                 preferred_element_type=jnp.float32)
    h = jnp.maximum(h + b1_ref[...], 0.0)
    h = jnp.maximum(
        jnp.dot(h, w2_ref[...], preferred_element_type=jnp.float32) + b2_ref[...],
        0.0)
    res = jnp.dot(h, wp_ref[...], preferred_element_type=jnp.float32) + bp_ref[...]
    out_ref[...] = jnp.transpose(res)


def _mlp(item_vec, ghl, gnl, pbr, W1a, W1b8, W1c8, W1pq, b1, W2, b2, Wp, bp,
         interpret=False):
    row = lambda i: (i, 0)
    rep = lambda i: (0, 0)
    return pl.pallas_call(
        _mlp_body,
        grid=(B // BLK,),
        in_specs=[
            pl.BlockSpec((BLK, D_ITEM), row),
            pl.BlockSpec((BLK, 128), row),
            pl.BlockSpec((BLK, 128), row),
            pl.BlockSpec((BLK, 4), row),
            pl.BlockSpec((D_ITEM, H1), rep),
            pl.BlockSpec((128, H1), rep),
            pl.BlockSpec((128, H1), rep),
            pl.BlockSpec((2, H1), rep),
            pl.BlockSpec((1, H1), rep),
            pl.BlockSpec((H1, H2), rep),
            pl.BlockSpec((1, H2), rep),
            pl.BlockSpec((H2, H2), rep),
            pl.BlockSpec((1, H2), rep),
        ],
        out_specs=pl.BlockSpec((H2, BLK), lambda i: (0, i)),
        out_shape=jax.ShapeDtypeStruct((H2, B), jnp.float32),
        interpret=interpret,
    )(item_vec, ghl, gnl, pbr, W1a, W1b8, W1c8, W1pq, b1, W2, b2, Wp, bp)


def kernel(ProductCode, ProductGroupHeader, ProductGroupName, Price, IsBestSeller,
           item_table, gh_table, gn_table, W1, b1, W2, b2, Wp, bp):
    gh128 = gh_table.reshape(125, 128)
    gn128 = gn_table.reshape(125, 128)
    ghl, gnl = _sc_groups()(
        ProductGroupHeader >> 3, ProductGroupName >> 3, gh128, gn128)
    item_vec = _sc_item()(ProductCode, item_table.reshape(125000, 8, D_ITEM))
    pbr = jnp.concatenate(
        [Price.astype(jnp.float32)[:, None],
         IsBestSeller.astype(jnp.float32)[:, None],
         (ProductGroupHeader & 7).astype(jnp.float32)[:, None],
         (ProductGroupName & 7).astype(jnp.float32)[:, None]], axis=1)
    W1b = W1[D_ITEM:D_ITEM + D_GROUP]
    W1c = W1[D_ITEM + D_GROUP:D_ITEM + 2 * D_GROUP]
    W1b8 = jnp.concatenate([W1b] * 8, axis=0)
    W1c8 = jnp.concatenate([W1c] * 8, axis=0)
    return _mlp(
        item_vec, ghl, gnl, pbr,
        W1[:D_ITEM], W1b8, W1c8, W1[D_ITEM + 2 * D_GROUP:],
        b1[None, :], W2, b2[None, :], Wp, bp[None, :]).T


# speedup vs baseline: 9.6705x; 1.0135x over previous
import jax
import jax.numpy as jnp
from jax import lax
from jax.experimental import pallas as pl
from jax.experimental.pallas import tpu as pltpu
from jax.experimental.pallas import tpu_sc as plsc

B = 16384
ITEM_VOCAB = 1000000
D_ITEM = 32
D_GROUP = 16
H1, H2 = 128, 64

NC, NS = 2, 16
NW = NC * NS
BPW = B // NW
CH = 256
G = 16
NG = BPW // G

BLK = 2048


def _sc_params():
    return dict(
        mesh=plsc.VectorSubcoreMesh(
            core_axis_name="c", subcore_axis_name="s",
            num_cores=NC, num_subcores=NS),
        compiler_params=pltpu.CompilerParams(use_tc_tiling_on_sc=True),
    )


def _group_body(ghl_hbm, gnl_hbm, gh128, gn128,
                ghl_out, gnl_out, lidx0, lidx1, lines0, lines1,
                sem0, sem1):
    wid = lax.axis_index("s") * NC + lax.axis_index("c")
    base = wid * BPW
    chunks = [(idx_hbm, src, out_hbm, base + ch * CH)
              for idx_hbm, src, out_hbm in ((ghl_hbm, gh128, ghl_out),
                                            (gnl_hbm, gn128, gnl_out))
              for ch in range(BPW // CH)]
    bufs = ((lidx0, lines0, sem0), (lidx1, lines1, sem1))
    prev = None
    for t, (idx_hbm, src, out_hbm, off) in enumerate(chunks):
        lidx, lines, sem = bufs[t % 2]
        pltpu.sync_copy(idx_hbm.at[pl.ds(off, CH)], lidx)
        c = pltpu.async_copy(src.at[lidx], lines, sem)
        if prev is not None:
            pc, plines, pout, poff = prev
            pc.wait()
            pltpu.sync_copy(plines, pout.at[pl.ds(poff, CH)])
        prev = (c, lines, out_hbm, off)
    pc, plines, pout, poff = prev
    pc.wait()
    pltpu.sync_copy(plines, pout.at[pl.ds(poff, CH)])


def _sc_groups():
    return pl.kernel(
        _group_body,
        out_type=[
            jax.ShapeDtypeStruct((B, 128), jnp.float32),
            jax.ShapeDtypeStruct((B, 128), jnp.float32),
        ],
        scratch_types=[
            pltpu.VMEM((CH,), jnp.int32),
            pltpu.VMEM((CH,), jnp.int32),
            pltpu.VMEM((CH, 128), jnp.float32),
            pltpu.VMEM((CH, 128), jnp.float32),
            pltpu.SemaphoreType.DMA,
            pltpu.SemaphoreType.DMA,
        ],
        **_sc_params(),
    )


def _item_body(pc_hbm, tbl, item_out, pc_v, slab0, slab1, rows_v,
               sem0, sem1):
    wid = lax.axis_index("s") * NC + lax.axis_index("c")
    base = wid * BPW
    pltpu.sync_copy(pc_hbm.at[pl.ds(base, BPW)], pc_v)

    def fire(gbase, slab, sem):
        pc16 = pc_v[pl.ds(gbase, G)]
        for k in range(G):
            rb = pc16[k] >> 3
            pltpu.async_copy(tbl.at[rb], slab.at[k], sem)
        return pc16

    def drain(slab, sem):
        for _ in range(G):
            pltpu.make_async_copy(tbl.at[0], slab.at[0], sem).wait()

    def extract(gbase, pc16, slab):
        for k in range(G):
            rem = jnp.bitwise_and(pc16[k], 7)
            for hh in range(2):
                rows_v[gbase + k, pl.ds(hh * 16, 16)] = (
                    slab[k, rem, pl.ds(hh * 16, 16)])

    fire(0, slab0, sem0)

    def pair(gp, carry):
        g0 = gp * 2 * G
        g1 = g0 + G
        pc0 = pc_v[pl.ds(g0, G)]
        fire(g1, slab1, sem1)
        drain(slab0, sem0)
        extract(g0, pc0, slab0)

        @pl.when(gp + 1 < NG // 2)
        def _():
            fire(g1 + G, slab0, sem0)

        pc1 = pc_v[pl.ds(g1, G)]
        drain(slab1, sem1)
        extract(g1, pc1, slab1)
        return carry

    lax.fori_loop(0, NG // 2, pair, 0)
    pltpu.sync_copy(rows_v, item_out.at[pl.ds(base, BPW)])


def _sc_item():
    return pl.kernel(
        _item_body,
        out_type=jax.ShapeDtypeStruct((B, D_ITEM), jnp.float32),
        scratch_types=[
            pltpu.VMEM((BPW,), jnp.int32),
            pltpu.VMEM((G, 8, D_ITEM), jnp.float32),
            pltpu.VMEM((G, 8, D_ITEM), jnp.float32),
            pltpu.VMEM((BPW, D_ITEM), jnp.float32),
            pltpu.SemaphoreType.DMA,
            pltpu.SemaphoreType.DMA,
        ],
        **_sc_params(),
    )


def _mlp_body(item_ref, ghl_ref, gnl_ref, pbr_ref,
              w1a_ref, w1b8_ref, w1c8_ref, w1pq_ref, b1_ref,
              w2_ref, b2_ref, wp_ref, bp_ref, out_ref):
    pbr = jnp.transpose(pbr_ref[...])
    h = jnp.dot(item_ref[...], w1a_ref[...], preferred_element_type=jnp.float32)
    seg8 = (lax.broadcasted_iota(jnp.int32, (BLK, 128), 1) >> 4).astype(
        jnp.float32)
    gh = jnp.where(seg8 == pbr[:, 2:3], ghl_ref[...], 0.0)
    gn = jnp.where(seg8 == pbr[:, 3:4], gnl_ref[...], 0.0)
    h += jnp.dot(gh, w1b8_ref[...], preferred_element_type=jnp.float32)
    h += jnp.dot(gn, w1c8_ref[...], preferred_element_type=jnp.float32)
    h += jnp.dot(pbr[:, 0:2], w1pq_ref[...],
                 preferred_element_type=jnp.float32)
    h = jnp.maximum(h + b1_ref[...], 0.0)
    h = jnp.maximum(
        jnp.dot(h, w2_ref[...], preferred_element_type=jnp.float32) + b2_ref[...],
        0.0)
    res = jnp.dot(h, wp_ref[...], preferred_element_type=jnp.float32) + bp_ref[...]
    out_ref[...] = jnp.transpose(res)


def _mlp(item_vec, ghl, gnl, pbr, W1a, W1b8, W1c8, W1pq, b1, W2, b2, Wp, bp,
         interpret=False):
    row = lambda i: (i, 0)
    rep = lambda i: (0, 0)
    return pl.pallas_call(
        _mlp_body,
        grid=(B // BLK,),
        in_specs=[
            pl.BlockSpec((BLK, D_ITEM), row),
            pl.BlockSpec((BLK, 128), row),
            pl.BlockSpec((BLK, 128), row),
            pl.BlockSpec((4, BLK), lambda i: (0, i)),
            pl.BlockSpec((D_ITEM, H1), rep),
            pl.BlockSpec((128, H1), rep),
            pl.BlockSpec((128, H1), rep),
            pl.BlockSpec((2, H1), rep),
            pl.BlockSpec((1, H1), rep),
            pl.BlockSpec((H1, H2), rep),
            pl.BlockSpec((1, H2), rep),
            pl.BlockSpec((H2, H2), rep),
            pl.BlockSpec((1, H2), rep),
        ],
        out_specs=pl.BlockSpec((H2, BLK), lambda i: (0, i)),
        out_shape=jax.ShapeDtypeStruct((H2, B), jnp.float32),
        interpret=interpret,
    )(item_vec, ghl, gnl, pbr, W1a, W1b8, W1c8, W1pq, b1, W2, b2, Wp, bp)


def kernel(ProductCode, ProductGroupHeader, ProductGroupName, Price, IsBestSeller,
           item_table, gh_table, gn_table, W1, b1, W2, b2, Wp, bp):
    gh128 = gh_table.reshape(125, 128)
    gn128 = gn_table.reshape(125, 128)
    ghl, gnl = _sc_groups()(
        ProductGroupHeader >> 3, ProductGroupName >> 3, gh128, gn128)
    item_vec = _sc_item()(ProductCode, item_table.reshape(125000, 8, D_ITEM))
    pbr = jnp.stack(
        [Price.astype(jnp.float32),
         IsBestSeller.astype(jnp.float32),
         (ProductGroupHeader & 7).astype(jnp.float32),
         (ProductGroupName & 7).astype(jnp.float32)], axis=0)
    W1b = W1[D_ITEM:D_ITEM + D_GROUP]
    W1c = W1[D_ITEM + D_GROUP:D_ITEM + 2 * D_GROUP]
    W1b8 = jnp.concatenate([W1b] * 8, axis=0)
    W1c8 = jnp.concatenate([W1c] * 8, axis=0)
    return _mlp(
        item_vec, ghl, gnl, pbr,
        W1[:D_ITEM], W1b8, W1c8, W1[D_ITEM + 2 * D_GROUP:],
        b1[None, :], W2, b2[None, :], Wp, bp[None, :]).T
